# baseline (speedup 1.0000x reference)
import functools
import math

import jax
import jax.numpy as jnp
from jax import lax
from jax.experimental import pallas as pl
from jax.experimental.pallas import tpu as pltpu


def _partial_loss_kernel(x_ref, t_ref, s_ref, out_ref, *, pack, d, c1, c2):
    diff = x_ref[...] - t_ref[...]
    sq = diff * diff

    s = s_ref[...]
    w = jnp.exp(-s)

    if pack > 1:
        lane_grp = lax.broadcasted_iota(jnp.int32, sq.shape, 1) // d
        we = w[:, 0:1]
        for p in range(1, pack):
            we = jnp.where(lane_grp == p, w[:, p : p + 1], we)
    else:
        we = w

    out_ref[0, 0, 0] = c1 * jnp.sum(sq * we) + c2 * jnp.sum(s)


def _mse_loss_opt(inputs, sigmas, targets, coef=1.0):
    D = inputs.shape[-1]
    R = math.prod(inputs.shape[:-1]) if inputs.ndim > 1 else 1

    x = inputs.reshape(R, D)
    t = targets.reshape(R, D)
    s = sigmas.reshape(R)

    if D < 128 and 128 % D == 0:
        P = 128 // D
    else:
        P = 1
    Dp = D * P
    Rp = -(-R // P)

    tr = max(8, min(2048, -(-Rp // 8) * 8))
    G = -(-Rp // tr)
    Rf = G * tr

    pad_rows = Rf * P - R
    if pad_rows:
        x = jnp.pad(x, ((0, pad_rows), (0, 0)))
        t = jnp.pad(t, ((0, pad_rows), (0, 0)))
        s = jnp.pad(s, ((0, pad_rows),))

    x = x.reshape(Rf, Dp)
    t = t.reshape(Rf, Dp)
    s = s.reshape(Rf, P)

    c1 = float(coef) * 0.5 / float(R * D)
    c2 = float(coef) * 0.5 / float(R)

    kernel_fn = functools.partial(_partial_loss_kernel, pack=P, d=D, c1=c1, c2=c2)

    partials = pl.pallas_call(
        kernel_fn,
        out_shape=jax.ShapeDtypeStruct((G, 1, 1), jnp.float32),
        grid=(G,),
        in_specs=[
            pl.BlockSpec((tr, Dp), lambda i: (i, 0)),
            pl.BlockSpec((tr, Dp), lambda i: (i, 0)),
            pl.BlockSpec((tr, P), lambda i: (i, 0)),
        ],
        out_specs=pl.BlockSpec(
            (1, 1, 1), lambda i: (i, 0, 0), memory_space=pltpu.SMEM
        ),
        compiler_params=pltpu.CompilerParams(
            dimension_semantics=("parallel",),
            vmem_limit_bytes=64 * 1024 * 1024,
        ),
    )(x, t, s)

    return jnp.sum(partials)


def kernel(inputs, sigmas, targets):
    return _mse_loss_opt(inputs, sigmas, targets, coef=1.0)
